# Initial kernel scaffold; baseline (speedup 1.0000x reference)
import jax
import jax.numpy as jnp
from jax.experimental import pallas as pl


def kernel(s, v, edge_index, edge_attr, rbf, fw1, fb1, fw2, fb2, uw1, ub1, uw2, ub2):
    raise NotImplementedError("write your pallas kernel here")


# speedup vs baseline: 5.2875x; 5.2875x over previous
import functools

import jax
import jax.numpy as jnp
from jax import lax
from jax.experimental import pallas as pl
from jax.experimental.pallas import tpu as pltpu
from jax.experimental.pallas import tpu_sc as plsc

N = 10000
E = 320000
F = 128
H = 64
NC = 2
NS = 16
EPT = E // NS
B = 40
NB = EPT // B
ROWS_PT = 624
ROWS_REM = N - NS * ROWS_PT

BT_PHI = 2000
BT_UPD = 1000


def _phi_body(rbf_ref, fw1_ref, fb1_ref, fw2_ref, fb2_ref, out_ref):
    h = jax.nn.silu(
        jnp.dot(rbf_ref[...], fw1_ref[...], preferred_element_type=jnp.float32)
        + fb1_ref[...]
    )
    phi = (
        jnp.dot(h, fw2_ref[...], preferred_element_type=jnp.float32)
        + fb2_ref[...]
    )
    for k in range(6):
        kind, half = k // 2, k % 2
        c0 = kind * F + half * H
        out_ref[k] = phi[:, c0:c0 + H]


def _run_phi(rbf, fw1, fb1, fw2, fb2):
    grid = E // BT_PHI
    return pl.pallas_call(
        _phi_body,
        grid=(grid,),
        in_specs=[
            pl.BlockSpec((BT_PHI, 20), lambda i: (i, 0)),
            pl.BlockSpec((20, F), lambda i: (0, 0)),
            pl.BlockSpec((F,), lambda i: (0,)),
            pl.BlockSpec((F, 3 * F), lambda i: (0, 0)),
            pl.BlockSpec((3 * F,), lambda i: (0,)),
        ],
        out_specs=pl.BlockSpec((6, BT_PHI, H), lambda i: (0, i, 0)),
        out_shape=jax.ShapeDtypeStruct((6, E, H), jnp.float32),
    )(rbf, fw1, fb1, fw2, fb2)


def _sc_body(phi6, t1, t2, dirb, row_h, col_h, out1, out2,
             accum, row_i, col_a, pss, pvv, psv, d1, d2,
             g1, g2, ob, sem):
    c = lax.axis_index("c")
    sid = lax.axis_index("s")
    e_base = sid * EPT
    r0 = sid * ROWS_PT

    def zero_own_rows():
        def _z(i, _):
            r = i // (F // 16)
            l = (i % (F // 16)) * 16
            ob[r, pl.ds(l, 16)] = jnp.zeros((16,), jnp.float32)
            return 0
        lax.fori_loop(0, B * (F // 16), _z, 0)
        for t in range(ROWS_PT // B):
            pltpu.sync_copy(ob, accum.at[pl.ds(r0 + t * B, B), :])
        rem = ROWS_PT % B
        if rem:
            pltpu.sync_copy(ob.at[pl.ds(0, rem), :],
                            accum.at[pl.ds(r0 + ROWS_PT - rem, rem), :])

        @pl.when(sid == NS - 1)
        def _():
            pltpu.sync_copy(ob.at[pl.ds(0, ROWS_REM), :],
                            accum.at[pl.ds(NS * ROWS_PT, ROWS_REM), :])

    def flush_own_rows(out):
        pltpu.sync_copy(accum.at[pl.ds(r0, ROWS_PT), :],
                        out.at[pl.ds(c * N + r0, ROWS_PT), :])

        @pl.when(sid == NS - 1)
        def _():
            pltpu.sync_copy(
                accum.at[pl.ds(NS * ROWS_PT, ROWS_REM), :],
                out.at[pl.ds(c * N + NS * ROWS_PT, ROWS_REM), :])

    def edge_block(b, pass2):
        e0 = e_base + b * B
        pltpu.sync_copy(row_h.at[pl.ds(e0, B)], row_i)
        pltpu.sync_copy(col_h.at[pl.ds(c * E + e0, B)], col_a)

        pltpu.async_copy(t1.at[col_a], g1, sem).wait()
        pltpu.sync_copy(phi6.at[pl.ds((4 + c) * E + e0, B), :], psv)
        pltpu.sync_copy(phi6.at[pl.ds((2 + c) * E + e0, B), :], pvv)
        if not pass2:
            pltpu.sync_copy(phi6.at[pl.ds(c * E + e0, B), :], pss)
            pltpu.sync_copy(dirb.at[pl.ds(e0, B), :], d1)
        else:
            pltpu.async_copy(t2.at[col_a], g2, sem).wait()
            pltpu.sync_copy(dirb.at[pl.ds(E + e0, B), :], d1)
            pltpu.sync_copy(dirb.at[pl.ds(2 * E + e0, B), :], d2)

        def _edge(e, _):
            dx = d1[e, :]
            if pass2:
                dy = d2[e, :]
            for j in range(H // 16):
                sl = pl.ds(j * 16, 16)
                sh = pl.ds(H + j * 16, 16)
                sv = g1[e, sl]
                t = psv[e, sl] * sv
                if not pass2:
                    ob[e, sl] = pss[e, sl] * sv
                    ob[e, sh] = pvv[e, sl] * g1[e, sh] + dx * t
                else:
                    ob[e, sl] = pvv[e, sl] * g2[e, sl] + dx * t
                    ob[e, sh] = pvv[e, sl] * g2[e, sh] + dy * t
            return 0
        lax.fori_loop(0, B, _edge, 0)
        pltpu.sync_copy(ob, accum.at[row_i], add=True)
        return jnp.int32(0)

    zero_own_rows()
    plsc.subcore_barrier()
    lax.fori_loop(0, NB, lambda b, x: edge_block(b, False), jnp.int32(0))
    plsc.subcore_barrier()
    flush_own_rows(out1)
    zero_own_rows()
    plsc.subcore_barrier()
    lax.fori_loop(0, NB, lambda b, x: edge_block(b, True), jnp.int32(0))
    plsc.subcore_barrier()
    flush_own_rows(out2)


def _run_sc(phi6, t1, t2, dirb, row, col):
    mesh = plsc.VectorSubcoreMesh(core_axis_name="c", subcore_axis_name="s")
    f32 = jnp.float32
    kern = pl.kernel(
        _sc_body,
        out_type=(
            jax.ShapeDtypeStruct((NC * N, F), f32),
            jax.ShapeDtypeStruct((NC * N, F), f32),
        ),
        mesh=mesh,
        scratch_types=[
            pltpu.VMEM_SHARED((N, F), f32),
            pltpu.VMEM((B,), jnp.int32),
            pltpu.VMEM((B,), jnp.int32),
            pltpu.VMEM((B, H), f32),
            pltpu.VMEM((B, H), f32),
            pltpu.VMEM((B, H), f32),
            pltpu.VMEM((B, 16), f32),
            pltpu.VMEM((B, 16), f32),
            pltpu.VMEM((B, F), f32),
            pltpu.VMEM((B, F), f32),
            pltpu.VMEM((B, F), f32),
            pltpu.SemaphoreType.DMA,
        ],
    )
    col2 = jnp.concatenate([col, col + N])
    return kern(phi6.reshape(6 * E, H), t1, t2, dirb.reshape(3 * E, 16),
                row, col2)


def _upd_body(s_ref, vt_ref, o1_ref, o2_ref, uw1_ref, ub1_ref, uw2_ref,
              ub2_ref, sout_ref, voutt_ref):
    sb = s_ref[...]
    ms = jnp.concatenate([o1_ref[0, :, 0:H], o1_ref[1, :, 0:H]], axis=1)
    mvx = jnp.concatenate([o1_ref[0, :, H:F], o1_ref[1, :, H:F]], axis=1)
    mvy = jnp.concatenate([o2_ref[0, :, 0:H], o2_ref[1, :, 0:H]], axis=1)
    mvz = jnp.concatenate([o2_ref[0, :, H:F], o2_ref[1, :, H:F]], axis=1)
    vn = jnp.sqrt(mvx * mvx + mvy * mvy + mvz * mvz)
    uw1 = uw1_ref[...]
    u1 = (
        jnp.dot(sb, uw1[0:F], preferred_element_type=jnp.float32)
        + jnp.dot(ms, uw1[F:2 * F], preferred_element_type=jnp.float32)
        + jnp.dot(vn, uw1[2 * F:3 * F], preferred_element_type=jnp.float32)
        + ub1_ref[...]
    )
    u = (
        jnp.dot(jax.nn.silu(u1), uw2_ref[...],
                preferred_element_type=jnp.float32)
        + ub2_ref[...]
    )
    sout_ref[...] = sb + u[:, 0:F]
    alpha = u[:, F:2 * F]
    beta = u[:, 2 * F:3 * F]
    voutt_ref[0] = alpha * vt_ref[0] + beta * mvx
    voutt_ref[1] = alpha * vt_ref[1] + beta * mvy
    voutt_ref[2] = alpha * vt_ref[2] + beta * mvz


def _run_update(s, vt, o1, o2, uw1, ub1, uw2, ub2):
    grid = N // BT_UPD
    return pl.pallas_call(
        _upd_body,
        grid=(grid,),
        in_specs=[
            pl.BlockSpec((BT_UPD, F), lambda i: (i, 0)),
            pl.BlockSpec((3, BT_UPD, F), lambda i: (0, i, 0)),
            pl.BlockSpec((2, BT_UPD, F), lambda i: (0, i, 0)),
            pl.BlockSpec((2, BT_UPD, F), lambda i: (0, i, 0)),
            pl.BlockSpec((3 * F, F), lambda i: (0, 0)),
            pl.BlockSpec((F,), lambda i: (0,)),
            pl.BlockSpec((F, 3 * F), lambda i: (0, 0)),
            pl.BlockSpec((3 * F,), lambda i: (0,)),
        ],
        out_specs=[
            pl.BlockSpec((BT_UPD, F), lambda i: (i, 0)),
            pl.BlockSpec((3, BT_UPD, F), lambda i: (0, i, 0)),
        ],
        out_shape=[
            jax.ShapeDtypeStruct((N, F), jnp.float32),
            jax.ShapeDtypeStruct((3, N, F), jnp.float32),
        ],
    )(s, vt, o1, o2, uw1, ub1, uw2, ub2)


def kernel(s, v, edge_index, edge_attr, rbf, fw1, fb1, fw2, fb2,
           uw1, ub1, uw2, ub2):
    row = edge_index[0].astype(jnp.int32)
    col = edge_index[1].astype(jnp.int32)

    phi6 = _run_phi(rbf, fw1, fb1, fw2, fb2)

    vt = jnp.transpose(v, (2, 0, 1))
    t1 = jnp.concatenate(
        [jnp.concatenate([s[:, h * H:(h + 1) * H],
                          vt[0, :, h * H:(h + 1) * H]], axis=1)
         for h in range(2)], axis=0)
    t2 = jnp.concatenate(
        [jnp.concatenate([vt[1, :, h * H:(h + 1) * H],
                          vt[2, :, h * H:(h + 1) * H]], axis=1)
         for h in range(2)], axis=0)
    dirs = edge_attr[:, 1:4]
    dirb = jnp.broadcast_to(
        jnp.transpose(dirs)[:, :, None], (3, E, 16)).astype(jnp.float32)

    o1, o2 = _run_sc(phi6, t1, t2, dirb, row, col)

    s_out, voutt = _run_update(
        s, vt, o1.reshape(2, N, F), o2.reshape(2, N, F), uw1, ub1, uw2, ub2)
    return s_out, jnp.transpose(voutt, (1, 2, 0))


# speedup vs baseline: 9.5754x; 1.8109x over previous
import functools

import jax
import jax.numpy as jnp
from jax import lax
from jax.experimental import pallas as pl
from jax.experimental.pallas import tpu as pltpu
from jax.experimental.pallas import tpu_sc as plsc

N = 10000
E = 320000
F = 128
H = 64
NC = 2
NS = 16
EPT = E // NS
B = 40
NB = EPT // B
ROWS_PT = 624
ROWS_REM = N - NS * ROWS_PT

BT_PHI = 2000
BT_UPD = 1000


def _phi_body(rbf_ref, fw1_ref, fb1_ref, fw2_ref, fb2_ref, out_ref):
    h = jax.nn.silu(
        jnp.dot(rbf_ref[...], fw1_ref[...], preferred_element_type=jnp.float32)
        + fb1_ref[...]
    )
    phi = (
        jnp.dot(h, fw2_ref[...], preferred_element_type=jnp.float32)
        + fb2_ref[...]
    )
    for k in range(6):
        kind, half = k // 2, k % 2
        c0 = kind * F + half * H
        out_ref[k] = phi[:, c0:c0 + H]


def _run_phi(rbf, fw1, fb1, fw2, fb2):
    grid = E // BT_PHI
    return pl.pallas_call(
        _phi_body,
        grid=(grid,),
        in_specs=[
            pl.BlockSpec((BT_PHI, 20), lambda i: (i, 0)),
            pl.BlockSpec((20, F), lambda i: (0, 0)),
            pl.BlockSpec((F,), lambda i: (0,)),
            pl.BlockSpec((F, 3 * F), lambda i: (0, 0)),
            pl.BlockSpec((3 * F,), lambda i: (0,)),
        ],
        out_specs=pl.BlockSpec((6, BT_PHI, H), lambda i: (0, i, 0)),
        out_shape=jax.ShapeDtypeStruct((6, E, H), jnp.float32),
    )(rbf, fw1, fb1, fw2, fb2)


def _sc_body(phi6, t1, t2, dirb, row_h, col_h, out1, out2,
             accum, row_i, col_a, pss, pvv, psv, d1, d2,
             g1, g2, ob, sem):
    c = lax.axis_index("c")
    sid = lax.axis_index("s")
    e_base = sid * EPT
    r0 = sid * ROWS_PT

    def zero_own_rows():
        def _z(i, _):
            r = i // (F // 16)
            l = (i % (F // 16)) * 16
            ob[r, pl.ds(l, 16)] = jnp.zeros((16,), jnp.float32)
            return 0
        lax.fori_loop(0, B * (F // 16), _z, 0)
        for t in range(ROWS_PT // B):
            pltpu.sync_copy(ob, accum.at[pl.ds(r0 + t * B, B), :])
        rem = ROWS_PT % B
        if rem:
            pltpu.sync_copy(ob.at[pl.ds(0, rem), :],
                            accum.at[pl.ds(r0 + ROWS_PT - rem, rem), :])

        @pl.when(sid == NS - 1)
        def _():
            pltpu.sync_copy(ob.at[pl.ds(0, ROWS_REM), :],
                            accum.at[pl.ds(NS * ROWS_PT, ROWS_REM), :])

    def flush_own_rows(out):
        pltpu.sync_copy(accum.at[pl.ds(r0, ROWS_PT), :],
                        out.at[pl.ds(c * N + r0, ROWS_PT), :])

        @pl.when(sid == NS - 1)
        def _():
            pltpu.sync_copy(
                accum.at[pl.ds(NS * ROWS_PT, ROWS_REM), :],
                out.at[pl.ds(c * N + NS * ROWS_PT, ROWS_REM), :])

    def edge_block(b, pass2):
        e0 = e_base + b * B
        hrow = pltpu.async_copy(row_h.at[pl.ds(e0, B)], row_i, sem)
        hcol = pltpu.async_copy(col_h.at[pl.ds(c * E + e0, B)], col_a, sem)
        hcol.wait()
        hs = [
            pltpu.async_copy(t1.at[col_a], g1, sem),
            pltpu.async_copy(phi6.at[pl.ds((4 + c) * E + e0, B), :], psv, sem),
            pltpu.async_copy(phi6.at[pl.ds((2 + c) * E + e0, B), :], pvv, sem),
        ]
        if not pass2:
            hs.append(pltpu.async_copy(
                phi6.at[pl.ds(c * E + e0, B), :], pss, sem))
            hs.append(pltpu.async_copy(dirb.at[pl.ds(e0, B), :], d1, sem))
        else:
            hs.append(pltpu.async_copy(t2.at[col_a], g2, sem))
            hs.append(pltpu.async_copy(dirb.at[pl.ds(E + e0, B), :], d1, sem))
            hs.append(pltpu.async_copy(
                dirb.at[pl.ds(2 * E + e0, B), :], d2, sem))
        for h in hs:
            h.wait()
        hrow.wait()

        def _edge(e, _):
            dx = d1[e, :]
            if pass2:
                dy = d2[e, :]
            for j in range(H // 16):
                sl = pl.ds(j * 16, 16)
                sh = pl.ds(H + j * 16, 16)
                sv = g1[e, sl]
                t = psv[e, sl] * sv
                if not pass2:
                    ob[e, sl] = pss[e, sl] * sv
                    ob[e, sh] = pvv[e, sl] * g1[e, sh] + dx * t
                else:
                    ob[e, sl] = pvv[e, sl] * g2[e, sl] + dx * t
                    ob[e, sh] = pvv[e, sl] * g2[e, sh] + dy * t
            return 0
        lax.fori_loop(0, B, _edge, 0)
        pltpu.sync_copy(ob, accum.at[row_i], add=True)
        return jnp.int32(0)

    zero_own_rows()
    plsc.subcore_barrier()
    lax.fori_loop(0, NB, lambda b, x: edge_block(b, False), jnp.int32(0))
    plsc.subcore_barrier()
    flush_own_rows(out1)
    zero_own_rows()
    plsc.subcore_barrier()
    lax.fori_loop(0, NB, lambda b, x: edge_block(b, True), jnp.int32(0))
    plsc.subcore_barrier()
    flush_own_rows(out2)


def _run_sc(phi6, t1, t2, dirb, row, col):
    mesh = plsc.VectorSubcoreMesh(core_axis_name="c", subcore_axis_name="s")
    f32 = jnp.float32
    kern = pl.kernel(
        _sc_body,
        out_type=(
            jax.ShapeDtypeStruct((NC * N, F), f32),
            jax.ShapeDtypeStruct((NC * N, F), f32),
        ),
        mesh=mesh,
        scratch_types=[
            pltpu.VMEM_SHARED((N, F), f32),
            pltpu.VMEM((B,), jnp.int32),
            pltpu.VMEM((B,), jnp.int32),
            pltpu.VMEM((B, H), f32),
            pltpu.VMEM((B, H), f32),
            pltpu.VMEM((B, H), f32),
            pltpu.VMEM((B, 16), f32),
            pltpu.VMEM((B, 16), f32),
            pltpu.VMEM((B, F), f32),
            pltpu.VMEM((B, F), f32),
            pltpu.VMEM((B, F), f32),
            pltpu.SemaphoreType.DMA,
        ],
    )
    col2 = jnp.concatenate([col, col + N])
    return kern(phi6.reshape(6 * E, H), t1, t2, dirb.reshape(3 * E, 16),
                row, col2)


def _upd_body(s_ref, vt_ref, o1_ref, o2_ref, uw1_ref, ub1_ref, uw2_ref,
              ub2_ref, sout_ref, voutt_ref):
    sb = s_ref[...]
    ms = jnp.concatenate([o1_ref[0, :, 0:H], o1_ref[1, :, 0:H]], axis=1)
    mvx = jnp.concatenate([o1_ref[0, :, H:F], o1_ref[1, :, H:F]], axis=1)
    mvy = jnp.concatenate([o2_ref[0, :, 0:H], o2_ref[1, :, 0:H]], axis=1)
    mvz = jnp.concatenate([o2_ref[0, :, H:F], o2_ref[1, :, H:F]], axis=1)
    vn = jnp.sqrt(mvx * mvx + mvy * mvy + mvz * mvz)
    uw1 = uw1_ref[...]
    u1 = (
        jnp.dot(sb, uw1[0:F], preferred_element_type=jnp.float32)
        + jnp.dot(ms, uw1[F:2 * F], preferred_element_type=jnp.float32)
        + jnp.dot(vn, uw1[2 * F:3 * F], preferred_element_type=jnp.float32)
        + ub1_ref[...]
    )
    u = (
        jnp.dot(jax.nn.silu(u1), uw2_ref[...],
                preferred_element_type=jnp.float32)
        + ub2_ref[...]
    )
    sout_ref[...] = sb + u[:, 0:F]
    alpha = u[:, F:2 * F]
    beta = u[:, 2 * F:3 * F]
    voutt_ref[0] = alpha * vt_ref[0] + beta * mvx
    voutt_ref[1] = alpha * vt_ref[1] + beta * mvy
    voutt_ref[2] = alpha * vt_ref[2] + beta * mvz


def _run_update(s, vt, o1, o2, uw1, ub1, uw2, ub2):
    grid = N // BT_UPD
    return pl.pallas_call(
        _upd_body,
        grid=(grid,),
        in_specs=[
            pl.BlockSpec((BT_UPD, F), lambda i: (i, 0)),
            pl.BlockSpec((3, BT_UPD, F), lambda i: (0, i, 0)),
            pl.BlockSpec((2, BT_UPD, F), lambda i: (0, i, 0)),
            pl.BlockSpec((2, BT_UPD, F), lambda i: (0, i, 0)),
            pl.BlockSpec((3 * F, F), lambda i: (0, 0)),
            pl.BlockSpec((F,), lambda i: (0,)),
            pl.BlockSpec((F, 3 * F), lambda i: (0, 0)),
            pl.BlockSpec((3 * F,), lambda i: (0,)),
        ],
        out_specs=[
            pl.BlockSpec((BT_UPD, F), lambda i: (i, 0)),
            pl.BlockSpec((3, BT_UPD, F), lambda i: (0, i, 0)),
        ],
        out_shape=[
            jax.ShapeDtypeStruct((N, F), jnp.float32),
            jax.ShapeDtypeStruct((3, N, F), jnp.float32),
        ],
    )(s, vt, o1, o2, uw1, ub1, uw2, ub2)


def kernel(s, v, edge_index, edge_attr, rbf, fw1, fb1, fw2, fb2,
           uw1, ub1, uw2, ub2):
    row = edge_index[0].astype(jnp.int32)
    col = edge_index[1].astype(jnp.int32)

    phi6 = _run_phi(rbf, fw1, fb1, fw2, fb2)

    vt = jnp.transpose(v, (2, 0, 1))
    t1 = jnp.concatenate(
        [jnp.concatenate([s[:, h * H:(h + 1) * H],
                          vt[0, :, h * H:(h + 1) * H]], axis=1)
         for h in range(2)], axis=0)
    t2 = jnp.concatenate(
        [jnp.concatenate([vt[1, :, h * H:(h + 1) * H],
                          vt[2, :, h * H:(h + 1) * H]], axis=1)
         for h in range(2)], axis=0)
    dirs = edge_attr[:, 1:4]
    dirb = jnp.broadcast_to(
        jnp.transpose(dirs)[:, :, None], (3, E, 16)).astype(jnp.float32)

    o1, o2 = _run_sc(phi6, t1, t2, dirb, row, col)

    s_out, voutt = _run_update(
        s, vt, o1.reshape(2, N, F), o2.reshape(2, N, F), uw1, ub1, uw2, ub2)
    return s_out, jnp.transpose(voutt, (1, 2, 0))


# speedup vs baseline: 9.5995x; 1.0025x over previous
import functools

import jax
import jax.numpy as jnp
from jax import lax
from jax.experimental import pallas as pl
from jax.experimental.pallas import tpu as pltpu
from jax.experimental.pallas import tpu_sc as plsc

N = 10000
E = 320000
F = 128
H = 64
NC = 2
NS = 16
EPT = E // NS
B = 40
NB = EPT // B
ROWS_PT = 624
ROWS_REM = N - NS * ROWS_PT

BT_PHI = 2000
BT_UPD = 1000


def _phi_body(rbf_ref, fw1_ref, fb1_ref, fw2_ref, fb2_ref, out_ref):
    h = jax.nn.silu(
        jnp.dot(rbf_ref[...], fw1_ref[...], preferred_element_type=jnp.float32)
        + fb1_ref[...]
    )
    phi = (
        jnp.dot(h, fw2_ref[...], preferred_element_type=jnp.float32)
        + fb2_ref[...]
    )
    for k in range(6):
        kind, half = k // 2, k % 2
        c0 = kind * F + half * H
        out_ref[k] = phi[:, c0:c0 + H]


def _run_phi(rbf, fw1, fb1, fw2, fb2):
    grid = E // BT_PHI
    return pl.pallas_call(
        _phi_body,
        grid=(grid,),
        in_specs=[
            pl.BlockSpec((BT_PHI, 20), lambda i: (i, 0)),
            pl.BlockSpec((20, F), lambda i: (0, 0)),
            pl.BlockSpec((F,), lambda i: (0,)),
            pl.BlockSpec((F, 3 * F), lambda i: (0, 0)),
            pl.BlockSpec((3 * F,), lambda i: (0,)),
        ],
        out_specs=pl.BlockSpec((6, BT_PHI, H), lambda i: (0, i, 0)),
        out_shape=jax.ShapeDtypeStruct((6, E, H), jnp.float32),
    )(rbf, fw1, fb1, fw2, fb2)


def _sc_body(phi6, t1, t2, dirb, row_h, col_h, out1, out2,
             accum, row_i, col_a, pss, pvv, psv, d1, d2,
             g1, g2, ob, sem, sem_idx):
    c = lax.axis_index("c")
    sid = lax.axis_index("s")
    e_base = sid * EPT
    r0 = sid * ROWS_PT

    def zero_own_rows():
        def _z(i, _):
            r = i // (F // 16)
            l = (i % (F // 16)) * 16
            ob[r, pl.ds(l, 16)] = jnp.zeros((16,), jnp.float32)
            return 0
        lax.fori_loop(0, B * (F // 16), _z, 0)
        for t in range(ROWS_PT // B):
            pltpu.sync_copy(ob, accum.at[pl.ds(r0 + t * B, B), :])
        rem = ROWS_PT % B
        if rem:
            pltpu.sync_copy(ob.at[pl.ds(0, rem), :],
                            accum.at[pl.ds(r0 + ROWS_PT - rem, rem), :])

        @pl.when(sid == NS - 1)
        def _():
            pltpu.sync_copy(ob.at[pl.ds(0, ROWS_REM), :],
                            accum.at[pl.ds(NS * ROWS_PT, ROWS_REM), :])

    def flush_own_rows(out):
        pltpu.sync_copy(accum.at[pl.ds(r0, ROWS_PT), :],
                        out.at[pl.ds(c * N + r0, ROWS_PT), :])

        @pl.when(sid == NS - 1)
        def _():
            pltpu.sync_copy(
                accum.at[pl.ds(NS * ROWS_PT, ROWS_REM), :],
                out.at[pl.ds(c * N + NS * ROWS_PT, ROWS_REM), :])

    def edge_block(b, pass2):
        e0 = e_base + b * B
        hrow = pltpu.async_copy(row_h.at[pl.ds(e0, B)], row_i, sem)
        hcol = pltpu.async_copy(col_h.at[pl.ds(c * E + e0, B)], col_a, sem_idx)
        hcol.wait()
        hs = [
            pltpu.async_copy(t1.at[col_a], g1, sem),
            pltpu.async_copy(phi6.at[pl.ds((4 + c) * E + e0, B), :], psv, sem),
            pltpu.async_copy(phi6.at[pl.ds((2 + c) * E + e0, B), :], pvv, sem),
        ]
        if not pass2:
            hs.append(pltpu.async_copy(
                phi6.at[pl.ds(c * E + e0, B), :], pss, sem))
            hs.append(pltpu.async_copy(dirb.at[pl.ds(e0, B), :], d1, sem))
        else:
            hs.append(pltpu.async_copy(t2.at[col_a], g2, sem))
            hs.append(pltpu.async_copy(dirb.at[pl.ds(E + e0, B), :], d1, sem))
            hs.append(pltpu.async_copy(
                dirb.at[pl.ds(2 * E + e0, B), :], d2, sem))
        for h in hs:
            h.wait()
        hrow.wait()

        def _edge(e, _):
            dx = d1[e, :]
            if pass2:
                dy = d2[e, :]
            for j in range(H // 16):
                sl = pl.ds(j * 16, 16)
                sh = pl.ds(H + j * 16, 16)
                sv = g1[e, sl]
                t = psv[e, sl] * sv
                if not pass2:
                    ob[e, sl] = pss[e, sl] * sv
                    ob[e, sh] = pvv[e, sl] * g1[e, sh] + dx * t
                else:
                    ob[e, sl] = pvv[e, sl] * g2[e, sl] + dx * t
                    ob[e, sh] = pvv[e, sl] * g2[e, sh] + dy * t
            return 0
        lax.fori_loop(0, B, _edge, 0)
        pltpu.sync_copy(ob, accum.at[row_i], add=True)
        return jnp.int32(0)

    zero_own_rows()
    plsc.subcore_barrier()
    lax.fori_loop(0, NB, lambda b, x: edge_block(b, False), jnp.int32(0))
    plsc.subcore_barrier()
    flush_own_rows(out1)
    zero_own_rows()
    plsc.subcore_barrier()
    lax.fori_loop(0, NB, lambda b, x: edge_block(b, True), jnp.int32(0))
    plsc.subcore_barrier()
    flush_own_rows(out2)


def _run_sc(phi6, t1, t2, dirb, row, col):
    mesh = plsc.VectorSubcoreMesh(core_axis_name="c", subcore_axis_name="s")
    f32 = jnp.float32
    kern = pl.kernel(
        _sc_body,
        out_type=(
            jax.ShapeDtypeStruct((NC * N, F), f32),
            jax.ShapeDtypeStruct((NC * N, F), f32),
        ),
        mesh=mesh,
        scratch_types=[
            pltpu.VMEM_SHARED((N, F), f32),
            pltpu.VMEM((B,), jnp.int32),
            pltpu.VMEM((B,), jnp.int32),
            pltpu.VMEM((B, H), f32),
            pltpu.VMEM((B, H), f32),
            pltpu.VMEM((B, H), f32),
            pltpu.VMEM((B, 16), f32),
            pltpu.VMEM((B, 16), f32),
            pltpu.VMEM((B, F), f32),
            pltpu.VMEM((B, F), f32),
            pltpu.VMEM((B, F), f32),
            pltpu.SemaphoreType.DMA,
            pltpu.SemaphoreType.DMA,
        ],
    )
    col2 = jnp.concatenate([col, col + N])
    return kern(phi6.reshape(6 * E, H), t1, t2, dirb.reshape(3 * E, 16),
                row, col2)


def _upd_body(s_ref, vt_ref, o1_ref, o2_ref, uw1_ref, ub1_ref, uw2_ref,
              ub2_ref, sout_ref, voutt_ref):
    sb = s_ref[...]
    ms = jnp.concatenate([o1_ref[0, :, 0:H], o1_ref[1, :, 0:H]], axis=1)
    mvx = jnp.concatenate([o1_ref[0, :, H:F], o1_ref[1, :, H:F]], axis=1)
    mvy = jnp.concatenate([o2_ref[0, :, 0:H], o2_ref[1, :, 0:H]], axis=1)
    mvz = jnp.concatenate([o2_ref[0, :, H:F], o2_ref[1, :, H:F]], axis=1)
    vn = jnp.sqrt(mvx * mvx + mvy * mvy + mvz * mvz)
    uw1 = uw1_ref[...]
    u1 = (
        jnp.dot(sb, uw1[0:F], preferred_element_type=jnp.float32)
        + jnp.dot(ms, uw1[F:2 * F], preferred_element_type=jnp.float32)
        + jnp.dot(vn, uw1[2 * F:3 * F], preferred_element_type=jnp.float32)
        + ub1_ref[...]
    )
    u = (
        jnp.dot(jax.nn.silu(u1), uw2_ref[...],
                preferred_element_type=jnp.float32)
        + ub2_ref[...]
    )
    sout_ref[...] = sb + u[:, 0:F]
    alpha = u[:, F:2 * F]
    beta = u[:, 2 * F:3 * F]
    voutt_ref[0] = alpha * vt_ref[0] + beta * mvx
    voutt_ref[1] = alpha * vt_ref[1] + beta * mvy
    voutt_ref[2] = alpha * vt_ref[2] + beta * mvz


def _run_update(s, vt, o1, o2, uw1, ub1, uw2, ub2):
    grid = N // BT_UPD
    return pl.pallas_call(
        _upd_body,
        grid=(grid,),
        in_specs=[
            pl.BlockSpec((BT_UPD, F), lambda i: (i, 0)),
            pl.BlockSpec((3, BT_UPD, F), lambda i: (0, i, 0)),
            pl.BlockSpec((2, BT_UPD, F), lambda i: (0, i, 0)),
            pl.BlockSpec((2, BT_UPD, F), lambda i: (0, i, 0)),
            pl.BlockSpec((3 * F, F), lambda i: (0, 0)),
            pl.BlockSpec((F,), lambda i: (0,)),
            pl.BlockSpec((F, 3 * F), lambda i: (0, 0)),
            pl.BlockSpec((3 * F,), lambda i: (0,)),
        ],
        out_specs=[
            pl.BlockSpec((BT_UPD, F), lambda i: (i, 0)),
            pl.BlockSpec((3, BT_UPD, F), lambda i: (0, i, 0)),
        ],
        out_shape=[
            jax.ShapeDtypeStruct((N, F), jnp.float32),
            jax.ShapeDtypeStruct((3, N, F), jnp.float32),
        ],
    )(s, vt, o1, o2, uw1, ub1, uw2, ub2)


def kernel(s, v, edge_index, edge_attr, rbf, fw1, fb1, fw2, fb2,
           uw1, ub1, uw2, ub2):
    row = edge_index[0].astype(jnp.int32)
    col = edge_index[1].astype(jnp.int32)

    phi6 = _run_phi(rbf, fw1, fb1, fw2, fb2)

    vt = jnp.transpose(v, (2, 0, 1))
    t1 = jnp.concatenate(
        [jnp.concatenate([s[:, h * H:(h + 1) * H],
                          vt[0, :, h * H:(h + 1) * H]], axis=1)
         for h in range(2)], axis=0)
    t2 = jnp.concatenate(
        [jnp.concatenate([vt[1, :, h * H:(h + 1) * H],
                          vt[2, :, h * H:(h + 1) * H]], axis=1)
         for h in range(2)], axis=0)
    dirs = edge_attr[:, 1:4]
    dirb = jnp.broadcast_to(
        jnp.transpose(dirs)[:, :, None], (3, E, 16)).astype(jnp.float32)

    o1, o2 = _run_sc(phi6, t1, t2, dirb, row, col)

    s_out, voutt = _run_update(
        s, vt, o1.reshape(2, N, F), o2.reshape(2, N, F), uw1, ub1, uw2, ub2)
    return s_out, jnp.transpose(voutt, (1, 2, 0))
